# baseline (speedup 1.0000x reference)
import jax
import jax.numpy as jnp
from jax import lax
from jax.experimental import pallas as pl
from jax.experimental.pallas import tpu as pltpu
from jax.experimental.pallas import tpu_sc as plsc

R = 65536
U = 4096
D = 256
C = 128
W = C + 8
NC, NS = 2, 16
NW = NC * NS
LANES = 16
GROUPS = D // LANES
BUF_SLOTS = 15
SLOTS = BUF_SLOTS + 1
NBUF = 3


def _mul8(x):
    return pl.multiple_of(x, 8)


def _scale_shift(buf, s, nrows):
    if isinstance(s, int) and s == 0:
        @plsc.parallel_loop(0, nrows, unroll=4)
        def _(l):
            for j in range(GROUPS):
                v = buf[l, pl.ds(j * LANES, LANES)]
                buf[l, pl.ds(j * LANES, LANES)] = v + v
        return

    def row(l, carry):
        for j in range(GROUPS):
            v = buf[l + s, pl.ds(j * LANES, LANES)]
            buf[l, pl.ds(j * LANES, LANES)] = v + v
        return carry
    lax.fori_loop(0, nrows, row, 0)


def _body(buf_hbm, upd_hbm, idx_hbm, out_hbm, idx_v, tiles, tile_a, tile_b,
          tile_t, sems_in, sems_out, sem_s):
    wid = lax.axis_index("s") * NC + lax.axis_index("c")

    pltpu.sync_copy(idx_hbm, idx_v.at[pl.ds(0, 2)])
    i0 = jnp.minimum(jnp.maximum(idx_v[pl.ds(0, LANES)][0], 0), R - U)

    m = i0 % 8
    i0f = _mul8(i0 - m)
    d1 = i0 + U
    d1f = _mul8(d1 - m)
    sh = (8 - m) % 8
    a0 = _mul8(i0 + sh)
    ab0 = _mul8(d1 + sh)

    t_full = i0f // C
    n_above = (R - ab0) // C
    hi0 = _mul8(R - n_above * C)
    n_bulk = t_full + n_above
    n_upd = (d1f - a0) // C
    nbt = (i0f % C) // 8
    nht = (hi0 - ab0) // 8
    nut = ((d1f - a0) % C) // 8
    n_mix = jnp.where(m != 0, 2, 0)
    n_small = nbt + nht + nut + n_mix

    src_ref, src_off, src_rows, dst_off, shift = ([None] * SLOTS for _ in range(5))
    for k in range(BUF_SLOTS):
        t = wid + NW * k
        t_eff = jnp.where(t < n_bulk, t, wid)
        off = _mul8(jnp.where(t_eff < t_full, t_eff * C,
                              hi0 + (t_eff - t_full) * C))
        src_ref[k], src_off[k], src_rows[k] = buf_hbm, off, C
        dst_off[k], shift[k] = off, 0
    w_eff = jnp.where(wid < n_upd, wid, 0)
    u0 = _mul8(jnp.minimum(w_eff * C, U - W))
    src_ref[-1], src_off[-1], src_rows[-1] = upd_hbm, u0, W
    dst_off[-1] = _mul8(a0 + w_eff * C)
    shift[-1] = sh + (w_eff * C - u0)

    def start_in(t):
        b = t % NBUF
        return pltpu.async_copy(
            src_ref[t].at[pl.ds(src_off[t], src_rows[t])],
            tiles[b].at[pl.ds(0, src_rows[t])], sems_in[b])

    def start_out(t):
        b = t % NBUF
        return pltpu.async_copy(
            tiles[b].at[pl.ds(0, C)],
            out_hbm.at[pl.ds(dst_off[t], C)], sems_out[b])

    in_d, out_d = [None] * SLOTS, [None] * SLOTS
    in_d[0] = start_in(0)
    in_d[1] = start_in(1)
    for t in range(SLOTS):
        in_d[t].wait()
        _scale_shift(tiles[t % NBUF], shift[t], C)
        out_d[t] = start_out(t)
        if t + 2 < SLOTS:
            if t - 1 >= 0:
                out_d[t - 1].wait()
            in_d[t + 2] = start_in(t + 2)
    out_d[SLOTS - 3].wait()
    out_d[SLOTS - 2].wait()
    out_d[SLOTS - 1].wait()

    def small_tile(j):
        pure = j < nbt + nht

        @pl.when(pure)
        def _():
            dst = _mul8(jnp.where(j < nbt, t_full * C + 8 * j,
                                  ab0 + 8 * (j - nbt)))
            pltpu.async_copy(buf_hbm.at[pl.ds(dst, 8)],
                             tile_a.at[pl.ds(0, 8)], sem_s).wait()
            _scale_shift(tile_a, 0, 8)
            pltpu.async_copy(tile_a.at[pl.ds(0, 8)],
                             out_hbm.at[pl.ds(dst, 8)], sem_s).wait()

        @pl.when((j >= nbt + nht) & (j < nbt + nht + nut))
        def _():
            jj = j - (nbt + nht)
            dst = _mul8(a0 + n_upd * C + 8 * jj)
            u = _mul8(n_upd * C + 8 * jj)
            pltpu.async_copy(upd_hbm.at[pl.ds(u, 16)], tile_a, sem_s).wait()
            _scale_shift(tile_a, sh, 8)
            pltpu.async_copy(tile_a.at[pl.ds(0, 8)],
                             out_hbm.at[pl.ds(dst, 8)], sem_s).wait()

        @pl.when((m != 0) & (j == nbt + nht + nut))
        def _():
            a_in = pltpu.async_copy(buf_hbm.at[pl.ds(i0f, 8)],
                                    tile_a.at[pl.ds(0, 8)], sem_s)
            b_in = pltpu.async_copy(upd_hbm.at[pl.ds(0, 8)],
                                    tile_b.at[pl.ds(0, 8)], sem_s)
            a_in.wait()
            b_in.wait()

            def row(l, carry):
                lb = jnp.maximum(l - m, 0)
                for g in range(GROUPS):
                    va = tile_a[l, pl.ds(g * LANES, LANES)]
                    vb = tile_b[lb, pl.ds(g * LANES, LANES)]
                    tile_t[l, pl.ds(g * LANES, LANES)] = jnp.where(
                        l >= m, vb + vb, va + va)
                return carry
            lax.fori_loop(0, 8, row, 0)
            pltpu.async_copy(tile_t.at[pl.ds(0, 8)],
                             out_hbm.at[pl.ds(i0f, 8)], sem_s).wait()

        @pl.when((m != 0) & (j == nbt + nht + nut + 1))
        def _():
            a_in = pltpu.async_copy(buf_hbm.at[pl.ds(d1f, 8)],
                                    tile_a.at[pl.ds(0, 8)], sem_s)
            b_in = pltpu.async_copy(upd_hbm.at[pl.ds(U - 8, 8)],
                                    tile_b.at[pl.ds(0, 8)], sem_s)
            a_in.wait()
            b_in.wait()

            def row(l, carry):
                lb = jnp.clip(8 - m + l, 0, 7)
                for g in range(GROUPS):
                    va = tile_a[l, pl.ds(g * LANES, LANES)]
                    vb = tile_b[lb, pl.ds(g * LANES, LANES)]
                    tile_t[l, pl.ds(g * LANES, LANES)] = jnp.where(
                        l < m, vb + vb, va + va)
                return carry
            lax.fori_loop(0, 8, row, 0)
            pltpu.async_copy(tile_t.at[pl.ds(0, 8)],
                             out_hbm.at[pl.ds(d1f, 8)], sem_s).wait()

    for k in range(2):
        j = wid + NW * k

        @pl.when(j < n_small)
        def _():
            small_tile(j)


@jax.jit
def kernel(buffer, update, index):
    mesh = plsc.VectorSubcoreMesh(core_axis_name="c", subcore_axis_name="s")
    return pl.kernel(
        _body,
        out_type=jax.ShapeDtypeStruct((R, D), jnp.float32),
        mesh=mesh,
        scratch_types=[
            pltpu.VMEM((LANES,), jnp.int32),
            [pltpu.VMEM((W, D), jnp.float32) for _ in range(NBUF)],
            pltpu.VMEM((16, D), jnp.float32),
            pltpu.VMEM((8, D), jnp.float32),
            pltpu.VMEM((8, D), jnp.float32),
            [pltpu.SemaphoreType.DMA for _ in range(NBUF)],
            [pltpu.SemaphoreType.DMA for _ in range(NBUF)],
            pltpu.SemaphoreType.DMA,
        ],
    )(buffer, update, index)


# speedup vs baseline: 1.0735x; 1.0735x over previous
import jax
import jax.numpy as jnp
from jax import lax
from jax.experimental import pallas as pl
from jax.experimental.pallas import tpu as pltpu
from jax.experimental.pallas import tpu_sc as plsc

R = 65536
U = 4096
D = 256
C = 128
W = C + 8
NC, NS = 2, 16
NW = NC * NS
LANES = 16
GROUPS = D // LANES
BUF_SLOTS = 15
SLOTS = BUF_SLOTS + 1
NBUF = 3


def _mul8(x):
    return pl.multiple_of(x, 8)


def _scale_shift(buf, s, nrows):
    def row(l, carry):
        for j in range(GROUPS):
            v = buf[l + s, pl.ds(j * LANES, LANES)]
            buf[l, pl.ds(j * LANES, LANES)] = v + v
        return carry
    lax.fori_loop(0, nrows, row, 0)


def _body(buf_hbm, upd_hbm, idx_hbm, out_hbm, idx_v, tiles, tile_a, tile_b,
          tile_t, sems_in, sems_out, sem_s):
    wid = lax.axis_index("s") * NC + lax.axis_index("c")

    pltpu.sync_copy(idx_hbm, idx_v.at[pl.ds(0, 2)])
    i0 = jnp.minimum(jnp.maximum(idx_v[pl.ds(0, LANES)][0], 0), R - U)

    m = i0 % 8
    i0f = _mul8(i0 - m)
    d1 = i0 + U
    d1f = _mul8(d1 - m)
    sh = (8 - m) % 8
    a0 = _mul8(i0 + sh)
    ab0 = _mul8(d1 + sh)

    t_full = i0f // C
    n_above = (R - ab0) // C
    hi0 = _mul8(R - n_above * C)
    n_bulk = t_full + n_above
    n_upd = (d1f - a0) // C
    nbt = (i0f % C) // 8
    nht = (hi0 - ab0) // 8
    nut = ((d1f - a0) % C) // 8
    n_mix = jnp.where(m != 0, 2, 0)
    n_small = nbt + nht + nut + n_mix

    src_ref, src_off, src_rows, dst_off, shift = ([None] * SLOTS for _ in range(5))
    for k in range(BUF_SLOTS):
        t = wid + NW * k
        t_eff = jnp.where(t < n_bulk, t, wid)
        off = _mul8(jnp.where(t_eff < t_full, t_eff * C,
                              hi0 + (t_eff - t_full) * C))
        src_ref[k], src_off[k], src_rows[k] = buf_hbm, off, C
        dst_off[k], shift[k] = off, 0
    w_eff = jnp.where(wid < n_upd, wid, 0)
    u0 = _mul8(jnp.minimum(w_eff * C, U - W))
    src_ref[-1], src_off[-1], src_rows[-1] = upd_hbm, u0, W
    dst_off[-1] = _mul8(a0 + w_eff * C)
    shift[-1] = sh + (w_eff * C - u0)

    def start_in(t):
        b = t % NBUF
        return pltpu.async_copy(
            src_ref[t].at[pl.ds(src_off[t], src_rows[t])],
            tiles[b].at[pl.ds(0, src_rows[t])], sems_in[b])

    def start_out(t):
        b = t % NBUF
        return pltpu.async_copy(
            tiles[b].at[pl.ds(0, C)],
            out_hbm.at[pl.ds(dst_off[t], C)], sems_out[b])

    in_d, out_d = [None] * SLOTS, [None] * SLOTS
    in_d[0] = start_in(0)
    in_d[1] = start_in(1)
    for t in range(SLOTS):
        in_d[t].wait()
        _scale_shift(tiles[t % NBUF], shift[t], C)
        out_d[t] = start_out(t)
        if t + 2 < SLOTS:
            if t - 1 >= 0:
                out_d[t - 1].wait()
            in_d[t + 2] = start_in(t + 2)
    out_d[SLOTS - 3].wait()
    out_d[SLOTS - 2].wait()
    out_d[SLOTS - 1].wait()

    def small_tile(j):
        pure = j < nbt + nht

        @pl.when(pure)
        def _():
            dst = _mul8(jnp.where(j < nbt, t_full * C + 8 * j,
                                  ab0 + 8 * (j - nbt)))
            pltpu.async_copy(buf_hbm.at[pl.ds(dst, 8)],
                             tile_a.at[pl.ds(0, 8)], sem_s).wait()
            _scale_shift(tile_a, 0, 8)
            pltpu.async_copy(tile_a.at[pl.ds(0, 8)],
                             out_hbm.at[pl.ds(dst, 8)], sem_s).wait()

        @pl.when((j >= nbt + nht) & (j < nbt + nht + nut))
        def _():
            jj = j - (nbt + nht)
            dst = _mul8(a0 + n_upd * C + 8 * jj)
            u = _mul8(n_upd * C + 8 * jj)
            pltpu.async_copy(upd_hbm.at[pl.ds(u, 16)], tile_a, sem_s).wait()
            _scale_shift(tile_a, sh, 8)
            pltpu.async_copy(tile_a.at[pl.ds(0, 8)],
                             out_hbm.at[pl.ds(dst, 8)], sem_s).wait()

        @pl.when((m != 0) & (j == nbt + nht + nut))
        def _():
            a_in = pltpu.async_copy(buf_hbm.at[pl.ds(i0f, 8)],
                                    tile_a.at[pl.ds(0, 8)], sem_s)
            b_in = pltpu.async_copy(upd_hbm.at[pl.ds(0, 8)],
                                    tile_b.at[pl.ds(0, 8)], sem_s)
            a_in.wait()
            b_in.wait()

            def row(l, carry):
                lb = jnp.maximum(l - m, 0)
                for g in range(GROUPS):
                    va = tile_a[l, pl.ds(g * LANES, LANES)]
                    vb = tile_b[lb, pl.ds(g * LANES, LANES)]
                    tile_t[l, pl.ds(g * LANES, LANES)] = jnp.where(
                        l >= m, vb + vb, va + va)
                return carry
            lax.fori_loop(0, 8, row, 0)
            pltpu.async_copy(tile_t.at[pl.ds(0, 8)],
                             out_hbm.at[pl.ds(i0f, 8)], sem_s).wait()

        @pl.when((m != 0) & (j == nbt + nht + nut + 1))
        def _():
            a_in = pltpu.async_copy(buf_hbm.at[pl.ds(d1f, 8)],
                                    tile_a.at[pl.ds(0, 8)], sem_s)
            b_in = pltpu.async_copy(upd_hbm.at[pl.ds(U - 8, 8)],
                                    tile_b.at[pl.ds(0, 8)], sem_s)
            a_in.wait()
            b_in.wait()

            def row(l, carry):
                lb = jnp.clip(8 - m + l, 0, 7)
                for g in range(GROUPS):
                    va = tile_a[l, pl.ds(g * LANES, LANES)]
                    vb = tile_b[lb, pl.ds(g * LANES, LANES)]
                    tile_t[l, pl.ds(g * LANES, LANES)] = jnp.where(
                        l < m, vb + vb, va + va)
                return carry
            lax.fori_loop(0, 8, row, 0)
            pltpu.async_copy(tile_t.at[pl.ds(0, 8)],
                             out_hbm.at[pl.ds(d1f, 8)], sem_s).wait()

    for k in range(2):
        j = wid + NW * k

        @pl.when(j < n_small)
        def _():
            small_tile(j)


@jax.jit
def kernel(buffer, update, index):
    mesh = plsc.VectorSubcoreMesh(core_axis_name="c", subcore_axis_name="s")
    return pl.kernel(
        _body,
        out_type=jax.ShapeDtypeStruct((R, D), jnp.float32),
        mesh=mesh,
        scratch_types=[
            pltpu.VMEM((LANES,), jnp.int32),
            [pltpu.VMEM((W, D), jnp.float32) for _ in range(NBUF)],
            pltpu.VMEM((16, D), jnp.float32),
            pltpu.VMEM((8, D), jnp.float32),
            pltpu.VMEM((8, D), jnp.float32),
            [pltpu.SemaphoreType.DMA for _ in range(NBUF)],
            [pltpu.SemaphoreType.DMA for _ in range(NBUF)],
            pltpu.SemaphoreType.DMA,
        ],
    )(buffer, update, index)


# speedup vs baseline: 1.0908x; 1.0162x over previous
import jax
import jax.numpy as jnp
from jax import lax
from jax.experimental import pallas as pl
from jax.experimental.pallas import tpu as pltpu
from jax.experimental.pallas import tpu_sc as plsc

R = 65536
U = 4096
D = 256
C = 128
W = C + 8
NC, NS = 2, 16
NW = NC * NS
LANES = 16
GROUPS = D // LANES
SLOTS = 16
NBUF = 3


def _mul8(x):
    return pl.multiple_of(x, 8)


def _scale_shift(buf, s, nrows):
    def row(l, carry):
        for j in range(GROUPS):
            v = buf[l + s, pl.ds(j * LANES, LANES)]
            buf[l, pl.ds(j * LANES, LANES)] = v + v
        return carry
    lax.fori_loop(0, nrows, row, 0)


def _body(buf_hbm, upd_hbm, idx_hbm, out_hbm, idx_v, tiles, sa, sb, st,
          sems_in, sems_out, sem_sr, sem_sw):
    wid = lax.axis_index("s") * NC + lax.axis_index("c")

    u0s = _mul8(jnp.minimum(wid * C, U - W))
    upd_in = pltpu.async_copy(upd_hbm.at[pl.ds(u0s, W)], tiles[0], sems_in[0])

    pltpu.sync_copy(idx_hbm, idx_v.at[pl.ds(0, 2)])
    i0 = jnp.minimum(jnp.maximum(idx_v[pl.ds(0, LANES)][0], 0), R - U)

    m = i0 % 8
    i0f = _mul8(i0 - m)
    d1 = i0 + U
    d1f = _mul8(d1 - m)
    sh = (8 - m) % 8
    a0 = _mul8(i0 + sh)
    ab0 = _mul8(d1 + sh)

    t_full = i0f // C
    n_above = (R - ab0) // C
    hi0 = _mul8(R - n_above * C)
    n_bulk = t_full + n_above
    n_upd = (d1f - a0) // C
    nbt = (i0f % C) // 8
    nht = (hi0 - ab0) // 8
    nut = ((d1f - a0) % C) // 8
    n_mix = jnp.where(m != 0, 2, 0)
    n_small = nbt + nht + nut + n_mix

    src_off, dst_off, shift = [None] * SLOTS, [None] * SLOTS, [None] * SLOTS
    dst_off[0] = _mul8(jnp.where(wid < n_upd, a0 + wid * C, i0 + (U - W) + sh))
    shift[0] = sh + jnp.where(wid < n_upd, wid * C - u0s, 0)
    for k in range(SLOTS - 1):
        t = wid + NW * k
        t_eff = jnp.where(t < n_bulk, t, wid)
        off = _mul8(jnp.where(t_eff < t_full, t_eff * C,
                              hi0 + (t_eff - t_full) * C))
        src_off[1 + k], dst_off[1 + k], shift[1 + k] = off, off, 0

    small = []
    for k in range(2):
        j = wid + NW * k
        is_pure = (j < nbt + nht) & (j < n_small)
        is_ut = (j >= nbt + nht) & (j < nbt + nht + nut)
        is_lo = (m != 0) & (j == nbt + nht + nut)
        is_hi = (m != 0) & (j == nbt + nht + nut + 1) & (j < n_small)
        p_dst = _mul8(jnp.where(j < nbt, t_full * C + 8 * j,
                                ab0 + 8 * (j - nbt)))
        u_src = _mul8(jnp.minimum(n_upd * C + 8 * (j - (nbt + nht)),
                                  U - 16))
        ut_dst = _mul8(a0 + n_upd * C + 8 * (j - (nbt + nht)))
        small.append((is_pure, is_ut, is_lo, is_hi, p_dst, u_src, ut_dst))

        @pl.when(is_pure)
        def _():
            pltpu.async_copy(buf_hbm.at[pl.ds(p_dst, 8)],
                             sa[k].at[pl.ds(0, 8)], sem_sr)

        @pl.when(is_ut)
        def _():
            pltpu.async_copy(upd_hbm.at[pl.ds(u_src, 16)], sa[k], sem_sr)

        @pl.when(is_lo)
        def _():
            pltpu.async_copy(buf_hbm.at[pl.ds(i0f, 8)],
                             sa[k].at[pl.ds(0, 8)], sem_sr)
            pltpu.async_copy(upd_hbm.at[pl.ds(0, 8)],
                             sb[k].at[pl.ds(0, 8)], sem_sr)

        @pl.when(is_hi)
        def _():
            pltpu.async_copy(buf_hbm.at[pl.ds(d1f, 8)],
                             sa[k].at[pl.ds(0, 8)], sem_sr)
            pltpu.async_copy(upd_hbm.at[pl.ds(U - 8, 8)],
                             sb[k].at[pl.ds(0, 8)], sem_sr)

    def start_in(t):
        return pltpu.async_copy(
            buf_hbm.at[pl.ds(src_off[t], C)],
            tiles[t % NBUF].at[pl.ds(0, C)], sems_in[t % NBUF])

    def start_out(t):
        return pltpu.async_copy(
            tiles[t % NBUF].at[pl.ds(0, C)],
            out_hbm.at[pl.ds(dst_off[t], C)], sems_out[t % NBUF])

    in_d, out_d = [None] * SLOTS, [None] * SLOTS
    in_d[0] = upd_in
    in_d[1] = start_in(1)
    in_d[2] = start_in(2)
    for t in range(SLOTS):
        in_d[t].wait()
        _scale_shift(tiles[t % NBUF], shift[t], C)
        out_d[t] = start_out(t)
        if 1 <= t <= SLOTS - 3:
            out_d[t - 1].wait()
            in_d[t + 2] = start_in(t + 2)

    sw = []
    for k in range(2):
        is_pure, is_ut, is_lo, is_hi, p_dst, u_src, ut_dst = small[k]

        @pl.when(is_pure)
        def _():
            pltpu.make_async_copy(buf_hbm.at[pl.ds(p_dst, 8)],
                                  sa[k].at[pl.ds(0, 8)], sem_sr).wait()
            _scale_shift(sa[k], 0, 8)
            pltpu.async_copy(sa[k].at[pl.ds(0, 8)],
                             out_hbm.at[pl.ds(p_dst, 8)], sem_sw)

        @pl.when(is_ut)
        def _():
            pltpu.make_async_copy(upd_hbm.at[pl.ds(u_src, 16)],
                                  sa[k], sem_sr).wait()
            _scale_shift(sa[k], ut_dst - i0 - u_src, 8)
            pltpu.async_copy(sa[k].at[pl.ds(0, 8)],
                             out_hbm.at[pl.ds(ut_dst, 8)], sem_sw)

        @pl.when(is_lo)
        def _():
            pltpu.make_async_copy(buf_hbm.at[pl.ds(i0f, 8)],
                                  sa[k].at[pl.ds(0, 8)], sem_sr).wait()
            pltpu.make_async_copy(upd_hbm.at[pl.ds(0, 8)],
                                  sb[k].at[pl.ds(0, 8)], sem_sr).wait()

            def row(l, carry):
                lb = jnp.maximum(l - m, 0)
                for g in range(GROUPS):
                    va = sa[k][l, pl.ds(g * LANES, LANES)]
                    vb = sb[k][lb, pl.ds(g * LANES, LANES)]
                    st[k][l, pl.ds(g * LANES, LANES)] = jnp.where(
                        l >= m, vb + vb, va + va)
                return carry
            lax.fori_loop(0, 8, row, 0)
            pltpu.async_copy(st[k].at[pl.ds(0, 8)],
                             out_hbm.at[pl.ds(i0f, 8)], sem_sw)

        @pl.when(is_hi)
        def _():
            pltpu.make_async_copy(buf_hbm.at[pl.ds(d1f, 8)],
                                  sa[k].at[pl.ds(0, 8)], sem_sr).wait()
            pltpu.make_async_copy(upd_hbm.at[pl.ds(U - 8, 8)],
                                  sb[k].at[pl.ds(0, 8)], sem_sr).wait()

            def row(l, carry):
                lb = jnp.clip(8 - m + l, 0, 7)
                for g in range(GROUPS):
                    va = sa[k][l, pl.ds(g * LANES, LANES)]
                    vb = sb[k][lb, pl.ds(g * LANES, LANES)]
                    st[k][l, pl.ds(g * LANES, LANES)] = jnp.where(
                        l < m, vb + vb, va + va)
                return carry
            lax.fori_loop(0, 8, row, 0)
            pltpu.async_copy(st[k].at[pl.ds(0, 8)],
                             out_hbm.at[pl.ds(d1f, 8)], sem_sw)

        sw.append((is_pure | is_ut, is_lo, is_hi, p_dst, ut_dst))

    out_d[SLOTS - 3].wait()
    out_d[SLOTS - 2].wait()
    out_d[SLOTS - 1].wait()
    for k in range(2):
        any_put, is_lo, is_hi, p_dst, ut_dst = sw[k]
        is_pure, is_ut = small[k][0], small[k][1]

        @pl.when(is_pure)
        def _():
            pltpu.make_async_copy(sa[k].at[pl.ds(0, 8)],
                                  out_hbm.at[pl.ds(p_dst, 8)], sem_sw).wait()

        @pl.when(is_ut)
        def _():
            pltpu.make_async_copy(sa[k].at[pl.ds(0, 8)],
                                  out_hbm.at[pl.ds(ut_dst, 8)], sem_sw).wait()

        @pl.when(is_lo)
        def _():
            pltpu.make_async_copy(st[k].at[pl.ds(0, 8)],
                                  out_hbm.at[pl.ds(i0f, 8)], sem_sw).wait()

        @pl.when(is_hi)
        def _():
            pltpu.make_async_copy(st[k].at[pl.ds(0, 8)],
                                  out_hbm.at[pl.ds(d1f, 8)], sem_sw).wait()


@jax.jit
def kernel(buffer, update, index):
    mesh = plsc.VectorSubcoreMesh(core_axis_name="c", subcore_axis_name="s")
    return pl.kernel(
        _body,
        out_type=jax.ShapeDtypeStruct((R, D), jnp.float32),
        mesh=mesh,
        scratch_types=[
            pltpu.VMEM((LANES,), jnp.int32),
            [pltpu.VMEM((W, D), jnp.float32),
             pltpu.VMEM((C, D), jnp.float32),
             pltpu.VMEM((C, D), jnp.float32)],
            [pltpu.VMEM((16, D), jnp.float32) for _ in range(2)],
            [pltpu.VMEM((8, D), jnp.float32) for _ in range(2)],
            [pltpu.VMEM((8, D), jnp.float32) for _ in range(2)],
            [pltpu.SemaphoreType.DMA for _ in range(NBUF)],
            [pltpu.SemaphoreType.DMA for _ in range(NBUF)],
            pltpu.SemaphoreType.DMA,
            pltpu.SemaphoreType.DMA,
        ],
    )(buffer, update, index)


# speedup vs baseline: 1.2135x; 1.1124x over previous
import jax
import jax.numpy as jnp
from jax import lax
from jax.experimental import pallas as pl
from jax.experimental.pallas import tpu as pltpu
from jax.experimental.pallas import tpu_sc as plsc

R = 65536
U = 4096
D = 256
C = 128
W = C + 8
NC, NS = 2, 16
NW = NC * NS
LANES = 16
GROUPS = D // LANES
SLOTS = 16
NBUF = 3


def _mul8(x):
    return pl.multiple_of(x, 8)


def _scale_shift(buf, s, nrows):
    def row(l, carry):
        vs = [buf[l + s, pl.ds(j * LANES, LANES)] for j in range(GROUPS)]
        for j in range(GROUPS):
            buf[l, pl.ds(j * LANES, LANES)] = vs[j] + vs[j]
        return carry
    lax.fori_loop(0, nrows, row, 0)


def _body(buf_hbm, upd_hbm, idx_hbm, out_hbm, idx_v, tiles, sa, sb, st,
          sems_in, sems_out, sem_sr, sem_sw):
    wid = lax.axis_index("s") * NC + lax.axis_index("c")

    u0s = _mul8(jnp.minimum(wid * C, U - W))
    upd_in = pltpu.async_copy(upd_hbm.at[pl.ds(u0s, W)], tiles[0], sems_in[0])

    pltpu.sync_copy(idx_hbm, idx_v.at[pl.ds(0, 2)])
    i0 = jnp.minimum(jnp.maximum(idx_v[pl.ds(0, LANES)][0], 0), R - U)

    m = i0 % 8
    i0f = _mul8(i0 - m)
    d1 = i0 + U
    d1f = _mul8(d1 - m)
    sh = (8 - m) % 8
    a0 = _mul8(i0 + sh)
    ab0 = _mul8(d1 + sh)

    t_full = i0f // C
    n_above = (R - ab0) // C
    hi0 = _mul8(R - n_above * C)
    n_bulk = t_full + n_above
    n_upd = (d1f - a0) // C
    nbt = (i0f % C) // 8
    nht = (hi0 - ab0) // 8
    nut = ((d1f - a0) % C) // 8
    n_mix = jnp.where(m != 0, 2, 0)
    n_small = nbt + nht + nut + n_mix

    src_off, dst_off, shift = [None] * SLOTS, [None] * SLOTS, [None] * SLOTS
    dst_off[0] = _mul8(jnp.where(wid < n_upd, a0 + wid * C, i0 + (U - W) + sh))
    shift[0] = sh + jnp.where(wid < n_upd, wid * C - u0s, 0)
    for k in range(SLOTS - 1):
        t = wid + NW * k
        t_eff = jnp.where(t < n_bulk, t, wid)
        off = _mul8(jnp.where(t_eff < t_full, t_eff * C,
                              hi0 + (t_eff - t_full) * C))
        src_off[1 + k], dst_off[1 + k], shift[1 + k] = off, off, 0

    small = []
    for k in range(2):
        j = wid + NW * k
        is_pure = (j < nbt + nht) & (j < n_small)
        is_ut = (j >= nbt + nht) & (j < nbt + nht + nut)
        is_lo = (m != 0) & (j == nbt + nht + nut)
        is_hi = (m != 0) & (j == nbt + nht + nut + 1) & (j < n_small)
        p_dst = _mul8(jnp.where(j < nbt, t_full * C + 8 * j,
                                ab0 + 8 * (j - nbt)))
        u_src = _mul8(jnp.minimum(n_upd * C + 8 * (j - (nbt + nht)),
                                  U - 16))
        ut_dst = _mul8(a0 + n_upd * C + 8 * (j - (nbt + nht)))
        small.append((is_pure, is_ut, is_lo, is_hi, p_dst, u_src, ut_dst))

        @pl.when(is_pure)
        def _():
            pltpu.async_copy(buf_hbm.at[pl.ds(p_dst, 8)],
                             sa[k].at[pl.ds(0, 8)], sem_sr)

        @pl.when(is_ut)
        def _():
            pltpu.async_copy(upd_hbm.at[pl.ds(u_src, 16)], sa[k], sem_sr)

        @pl.when(is_lo)
        def _():
            pltpu.async_copy(buf_hbm.at[pl.ds(i0f, 8)],
                             sa[k].at[pl.ds(0, 8)], sem_sr)
            pltpu.async_copy(upd_hbm.at[pl.ds(0, 8)],
                             sb[k].at[pl.ds(0, 8)], sem_sr)

        @pl.when(is_hi)
        def _():
            pltpu.async_copy(buf_hbm.at[pl.ds(d1f, 8)],
                             sa[k].at[pl.ds(0, 8)], sem_sr)
            pltpu.async_copy(upd_hbm.at[pl.ds(U - 8, 8)],
                             sb[k].at[pl.ds(0, 8)], sem_sr)

    def start_in(t):
        return pltpu.async_copy(
            buf_hbm.at[pl.ds(src_off[t], C)],
            tiles[t % NBUF].at[pl.ds(0, C)], sems_in[t % NBUF])

    def start_out(t):
        return pltpu.async_copy(
            tiles[t % NBUF].at[pl.ds(0, C)],
            out_hbm.at[pl.ds(dst_off[t], C)], sems_out[t % NBUF])

    in_d, out_d = [None] * SLOTS, [None] * SLOTS
    in_d[0] = upd_in
    in_d[1] = start_in(1)
    in_d[2] = start_in(2)
    for t in range(SLOTS):
        in_d[t].wait()
        _scale_shift(tiles[t % NBUF], shift[t], C)
        out_d[t] = start_out(t)
        if 1 <= t <= SLOTS - 3:
            out_d[t - 1].wait()
            in_d[t + 2] = start_in(t + 2)

    sw = []
    for k in range(2):
        is_pure, is_ut, is_lo, is_hi, p_dst, u_src, ut_dst = small[k]

        @pl.when(is_pure)
        def _():
            pltpu.make_async_copy(buf_hbm.at[pl.ds(p_dst, 8)],
                                  sa[k].at[pl.ds(0, 8)], sem_sr).wait()
            _scale_shift(sa[k], 0, 8)
            pltpu.async_copy(sa[k].at[pl.ds(0, 8)],
                             out_hbm.at[pl.ds(p_dst, 8)], sem_sw)

        @pl.when(is_ut)
        def _():
            pltpu.make_async_copy(upd_hbm.at[pl.ds(u_src, 16)],
                                  sa[k], sem_sr).wait()
            _scale_shift(sa[k], ut_dst - i0 - u_src, 8)
            pltpu.async_copy(sa[k].at[pl.ds(0, 8)],
                             out_hbm.at[pl.ds(ut_dst, 8)], sem_sw)

        @pl.when(is_lo)
        def _():
            pltpu.make_async_copy(buf_hbm.at[pl.ds(i0f, 8)],
                                  sa[k].at[pl.ds(0, 8)], sem_sr).wait()
            pltpu.make_async_copy(upd_hbm.at[pl.ds(0, 8)],
                                  sb[k].at[pl.ds(0, 8)], sem_sr).wait()

            def row(l, carry):
                lb = jnp.maximum(l - m, 0)
                for g in range(GROUPS):
                    va = sa[k][l, pl.ds(g * LANES, LANES)]
                    vb = sb[k][lb, pl.ds(g * LANES, LANES)]
                    st[k][l, pl.ds(g * LANES, LANES)] = jnp.where(
                        l >= m, vb + vb, va + va)
                return carry
            lax.fori_loop(0, 8, row, 0)
            pltpu.async_copy(st[k].at[pl.ds(0, 8)],
                             out_hbm.at[pl.ds(i0f, 8)], sem_sw)

        @pl.when(is_hi)
        def _():
            pltpu.make_async_copy(buf_hbm.at[pl.ds(d1f, 8)],
                                  sa[k].at[pl.ds(0, 8)], sem_sr).wait()
            pltpu.make_async_copy(upd_hbm.at[pl.ds(U - 8, 8)],
                                  sb[k].at[pl.ds(0, 8)], sem_sr).wait()

            def row(l, carry):
                lb = jnp.clip(8 - m + l, 0, 7)
                for g in range(GROUPS):
                    va = sa[k][l, pl.ds(g * LANES, LANES)]
                    vb = sb[k][lb, pl.ds(g * LANES, LANES)]
                    st[k][l, pl.ds(g * LANES, LANES)] = jnp.where(
                        l < m, vb + vb, va + va)
                return carry
            lax.fori_loop(0, 8, row, 0)
            pltpu.async_copy(st[k].at[pl.ds(0, 8)],
                             out_hbm.at[pl.ds(d1f, 8)], sem_sw)

        sw.append((is_pure | is_ut, is_lo, is_hi, p_dst, ut_dst))

    out_d[SLOTS - 3].wait()
    out_d[SLOTS - 2].wait()
    out_d[SLOTS - 1].wait()
    for k in range(2):
        any_put, is_lo, is_hi, p_dst, ut_dst = sw[k]
        is_pure, is_ut = small[k][0], small[k][1]

        @pl.when(is_pure)
        def _():
            pltpu.make_async_copy(sa[k].at[pl.ds(0, 8)],
                                  out_hbm.at[pl.ds(p_dst, 8)], sem_sw).wait()

        @pl.when(is_ut)
        def _():
            pltpu.make_async_copy(sa[k].at[pl.ds(0, 8)],
                                  out_hbm.at[pl.ds(ut_dst, 8)], sem_sw).wait()

        @pl.when(is_lo)
        def _():
            pltpu.make_async_copy(st[k].at[pl.ds(0, 8)],
                                  out_hbm.at[pl.ds(i0f, 8)], sem_sw).wait()

        @pl.when(is_hi)
        def _():
            pltpu.make_async_copy(st[k].at[pl.ds(0, 8)],
                                  out_hbm.at[pl.ds(d1f, 8)], sem_sw).wait()


@jax.jit
def kernel(buffer, update, index):
    mesh = plsc.VectorSubcoreMesh(core_axis_name="c", subcore_axis_name="s")
    return pl.kernel(
        _body,
        out_type=jax.ShapeDtypeStruct((R, D), jnp.float32),
        mesh=mesh,
        scratch_types=[
            pltpu.VMEM((LANES,), jnp.int32),
            [pltpu.VMEM((W, D), jnp.float32),
             pltpu.VMEM((C, D), jnp.float32),
             pltpu.VMEM((C, D), jnp.float32)],
            [pltpu.VMEM((16, D), jnp.float32) for _ in range(2)],
            [pltpu.VMEM((8, D), jnp.float32) for _ in range(2)],
            [pltpu.VMEM((8, D), jnp.float32) for _ in range(2)],
            [pltpu.SemaphoreType.DMA for _ in range(NBUF)],
            [pltpu.SemaphoreType.DMA for _ in range(NBUF)],
            pltpu.SemaphoreType.DMA,
            pltpu.SemaphoreType.DMA,
        ],
    )(buffer, update, index)
